# Initial kernel scaffold; baseline (speedup 1.0000x reference)
import jax
import jax.numpy as jnp
from jax.experimental import pallas as pl


def kernel(x, pe):
    raise NotImplementedError("write your pallas kernel here")


# speedup vs baseline: 2.7498x; 2.7498x over previous
import jax
import jax.numpy as jnp
from jax.experimental import pallas as pl


def _copy_body(pe_ref, out_ref):
    out_ref[...] = pe_ref[...]


def kernel(x, pe):
    seq_len = x.shape[1]
    d = pe.shape[1]
    blk = 512
    out = pl.pallas_call(
        _copy_body,
        grid=(seq_len // blk,),
        in_specs=[pl.BlockSpec((blk, d), lambda i: (i, 0))],
        out_specs=pl.BlockSpec((blk, d), lambda i: (i, 0)),
        out_shape=jax.ShapeDtypeStruct((seq_len, d), jnp.float32),
    )(pe)
    return out[None]
